# baseline (speedup 1.0000x reference)
import functools

import jax
import jax.numpy as jnp
from jax import lax
from jax.experimental import pallas as pl
from jax.experimental.pallas import tpu as pltpu
from jax.experimental.pallas import tpu_sc as plsc

VOCAB = 50257
SEQ = 2048
EMBED = 768
BATCH = 4

_INFO = plsc.get_sparse_core_info()
NC = _INFO.num_cores
NS = _INFO.num_subcores
L = _INFO.num_lanes
NW = NC * NS
S_PER_W = SEQ // NW
VPR = EMBED // L

CH = 16
NBUF = 6
LA = 4
NCHUNK = BATCH * S_PER_W // CH
CPB = S_PER_W // CH


@functools.partial(
    pl.kernel,
    out_type=jax.ShapeDtypeStruct((BATCH, SEQ, EMBED), jnp.float32),
    mesh=plsc.VectorSubcoreMesh(core_axis_name="c", subcore_axis_name="s"),
    scratch_types=[
        pltpu.VMEM((BATCH, S_PER_W), jnp.int32),
        pltpu.VMEM((S_PER_W, EMBED), jnp.float32),
        pltpu.VMEM((NBUF, CH, EMBED), jnp.float32),
        pltpu.SemaphoreType.DMA((NBUF,)),
        pltpu.SemaphoreType.DMA((NBUF,)),
        pltpu.SemaphoreType.DMA,
    ],
)
def _emb_kernel(
    ids_hbm, table_hbm, pos_hbm, out_hbm, idx_v, pos_v, rows_v, gsem, wsem, psem
):
    wid = lax.axis_index("s") * NC + lax.axis_index("c")
    s_base = wid * S_PER_W

    pltpu.sync_copy(ids_hbm.at[0, pl.ds(s_base, S_PER_W)], idx_v.at[0])

    def start_gather(t, buf):
        b = t // CPB
        h = t - b * CPB
        pltpu.async_copy(
            table_hbm.at[idx_v.at[b, pl.ds(h * CH, CH)]],
            rows_v.at[buf],
            gsem.at[buf],
        )

    def wait_gather(buf):
        pltpu.make_async_copy(
            table_hbm.at[idx_v.at[0, pl.ds(0, CH)]],
            rows_v.at[buf],
            gsem.at[buf],
        ).wait()

    def wait_wb(buf):
        pltpu.make_async_copy(
            rows_v.at[buf],
            out_hbm.at[0, pl.ds(0, CH)],
            wsem.at[buf],
        ).wait()

    for t0 in range(LA):
        start_gather(t0, t0)
    stage = [
        pltpu.async_copy(ids_hbm.at[b, pl.ds(s_base, S_PER_W)], idx_v.at[b], psem)
        for b in range(1, BATCH)
    ]
    stage.append(pltpu.async_copy(pos_hbm.at[pl.ds(s_base, S_PER_W)], pos_v, psem))
    for cp in stage:
        cp.wait()

    def chunk_body(t, _):
        buf = lax.rem(t, NBUF)
        b = t // CPB
        h = t - b * CPB

        nxt = t + LA

        @pl.when(nxt < NCHUNK)
        def _():
            nbuf = lax.rem(nxt, NBUF)

            @pl.when(nxt >= NBUF)
            def _():
                wait_wb(nbuf)

            start_gather(nxt, nbuf)

        wait_gather(buf)

        @plsc.parallel_loop(0, CH)
        def _add_row(i):
            for j in range(VPR):
                plsc.addupdate(
                    rows_v.at[buf, i, pl.ds(j * L, L)],
                    pos_v[h * CH + i, pl.ds(j * L, L)],
                )

        pltpu.async_copy(
            rows_v.at[buf],
            out_hbm.at[b, pl.ds(s_base + h * CH, CH)],
            wsem.at[buf],
        )
        return 0

    lax.fori_loop(0, NCHUNK, chunk_body, 0)

    for t in range(NCHUNK - NBUF, NCHUNK):
        wait_wb(t % NBUF)


def kernel(input_ids, token_embeddings, position_embeddings):
    return _emb_kernel(input_ids, token_embeddings, position_embeddings)


# speedup vs baseline: 1.0084x; 1.0084x over previous
import functools

import jax
import jax.numpy as jnp
from jax import lax
from jax.experimental import pallas as pl
from jax.experimental.pallas import tpu as pltpu
from jax.experimental.pallas import tpu_sc as plsc

VOCAB = 50257
SEQ = 2048
EMBED = 768
BATCH = 4

_INFO = plsc.get_sparse_core_info()
NC = _INFO.num_cores
NS = _INFO.num_subcores
L = _INFO.num_lanes
NW = NC * NS
S_PER_W = SEQ // NW
VPR = EMBED // L

CH = 16
NBUF = 6
LA = 3
NCHUNK = BATCH * S_PER_W // CH
CPB = S_PER_W // CH


@functools.partial(
    pl.kernel,
    out_type=jax.ShapeDtypeStruct((BATCH, SEQ, EMBED), jnp.float32),
    mesh=plsc.VectorSubcoreMesh(core_axis_name="c", subcore_axis_name="s"),
    scratch_types=[
        pltpu.VMEM((BATCH, S_PER_W), jnp.int32),
        pltpu.VMEM((S_PER_W, EMBED), jnp.float32),
        pltpu.VMEM((NBUF, CH, EMBED), jnp.float32),
        pltpu.SemaphoreType.DMA((NBUF,)),
        pltpu.SemaphoreType.DMA((NBUF,)),
        pltpu.SemaphoreType.DMA,
    ],
)
def _emb_kernel(
    ids_hbm, table_hbm, pos_hbm, out_hbm, idx_v, pos_v, rows_v, gsem, wsem, psem
):
    wid = lax.axis_index("s") * NC + lax.axis_index("c")
    s_base = wid * S_PER_W

    pltpu.sync_copy(ids_hbm.at[0, pl.ds(s_base, S_PER_W)], idx_v.at[0])

    def start_gather(t, buf):
        b = t // CPB
        h = t - b * CPB
        pltpu.async_copy(
            table_hbm.at[idx_v.at[b, pl.ds(h * CH, CH)]],
            rows_v.at[buf],
            gsem.at[buf],
        )

    def wait_gather(buf):
        pltpu.make_async_copy(
            table_hbm.at[idx_v.at[0, pl.ds(0, CH)]],
            rows_v.at[buf],
            gsem.at[buf],
        ).wait()

    def wait_wb(buf):
        pltpu.make_async_copy(
            rows_v.at[buf],
            out_hbm.at[0, pl.ds(0, CH)],
            wsem.at[buf],
        ).wait()

    for t0 in range(LA):
        start_gather(t0, t0)
    stage = [
        pltpu.async_copy(ids_hbm.at[b, pl.ds(s_base, S_PER_W)], idx_v.at[b], psem)
        for b in range(1, BATCH)
    ]
    stage.append(pltpu.async_copy(pos_hbm.at[pl.ds(s_base, S_PER_W)], pos_v, psem))
    for cp in stage:
        cp.wait()

    def chunk_body(t, _):
        buf = lax.rem(t, NBUF)
        b = t // CPB
        h = t - b * CPB

        nxt = t + LA

        @pl.when(nxt < NCHUNK)
        def _():
            nbuf = lax.rem(nxt, NBUF)

            @pl.when(nxt >= NBUF)
            def _():
                wait_wb(nbuf)

            start_gather(nxt, nbuf)

        wait_gather(buf)

        @plsc.parallel_loop(0, CH)
        def _add_row(i):
            for j in range(VPR):
                plsc.addupdate(
                    rows_v.at[buf, i, pl.ds(j * L, L)],
                    pos_v[h * CH + i, pl.ds(j * L, L)],
                )

        pltpu.async_copy(
            rows_v.at[buf],
            out_hbm.at[b, pl.ds(s_base + h * CH, CH)],
            wsem.at[buf],
        )
        return 0

    lax.fori_loop(0, NCHUNK, chunk_body, 0)

    for t in range(NCHUNK - NBUF, NCHUNK):
        wait_wb(t % NBUF)


def kernel(input_ids, token_embeddings, position_embeddings):
    return _emb_kernel(input_ids, token_embeddings, position_embeddings)
